# baseline (speedup 1.0000x reference)
import functools

import jax
import jax.numpy as jnp
from jax import lax
from jax.experimental import pallas as pl
from jax.experimental.pallas import tpu as pltpu
from jax.experimental.pallas import tpu_sc as plsc

_LR_SCALE = 10.0
_WEIGHT_SCALE = 0.2

_NC = 2
_NS = 16
_LANES = 16
_NW = _NC * _NS


def _sc_lookup_scale(weight):
    f_dim, c_dim = weight.shape
    rows_pw = f_dim // _NW
    per_row = c_dim // _LANES
    scale = _LR_SCALE * _WEIGHT_SCALE

    def body(w_hbm, out_hbm, v):
        wid = lax.axis_index("s") * _NC + lax.axis_index("c")
        base = wid * rows_pw
        pltpu.sync_copy(w_hbm.at[pl.ds(base, rows_pw), :], v)
        for r in range(rows_pw):
            for k in range(per_row):
                sl = pl.ds(k * _LANES, _LANES)
                v[r, sl] = v[r, sl] * scale
        pltpu.sync_copy(v, out_hbm.at[pl.ds(base, rows_pw), :])

    mesh = plsc.VectorSubcoreMesh(core_axis_name="c", subcore_axis_name="s")
    fn = functools.partial(
        pl.kernel,
        mesh=mesh,
        out_type=jax.ShapeDtypeStruct((f_dim, c_dim), jnp.float32),
        scratch_types=[pltpu.VMEM((rows_pw, c_dim), jnp.float32)],
    )(body)
    return fn(weight)


_NGRP = 8


def _tc_expand(tbl, batch, f_dim, c_dim, t_dim):
    rows = c_dim * f_dim
    cpg = c_dim // _NGRP

    def body(t_ref, out_ref, scratch_ref, sem):
        t = t_ref[...]
        copies = []
        for g in range(_NGRP):
            for c in range(g * cpg, (g + 1) * cpg):
                col = lax.slice(t, (0, c), (f_dim, c + 1))
                scratch_ref[pl.ds(c * f_dim, f_dim), :] = lax.broadcast_in_dim(
                    col, (f_dim, t_dim), (0, 1)
                )
            sl = pl.ds(g * cpg * f_dim, cpg * f_dim)
            for b in range(batch):
                cp = pltpu.make_async_copy(
                    scratch_ref.at[sl, :], out_ref.at[b, sl, :], sem
                )
                cp.start()
                copies.append(cp)
        for cp in copies:
            cp.wait()

    return pl.pallas_call(
        body,
        in_specs=[pl.BlockSpec(memory_space=pltpu.VMEM)],
        out_specs=pl.BlockSpec(memory_space=pl.ANY),
        out_shape=jax.ShapeDtypeStruct((batch, rows, t_dim), jnp.float32),
        scratch_shapes=[
            pltpu.VMEM((rows, t_dim), jnp.float32),
            pltpu.SemaphoreType.DMA,
        ],
        compiler_params=pltpu.CompilerParams(
            vmem_limit_bytes=100 * 1024 * 1024,
        ),
    )(tbl)


def kernel(input_features, weight):
    batch, c_dim, f_dim, t_dim = input_features.shape
    tbl = _sc_lookup_scale(weight)
    out3 = _tc_expand(tbl, batch, f_dim, c_dim, t_dim)
    return out3.reshape(batch, c_dim, f_dim, t_dim)


# speedup vs baseline: 1.0125x; 1.0125x over previous
import functools

import jax
import jax.numpy as jnp
from jax import lax
from jax.experimental import pallas as pl
from jax.experimental.pallas import tpu as pltpu
from jax.experimental.pallas import tpu_sc as plsc

_LR_SCALE = 10.0
_WEIGHT_SCALE = 0.2

_NC = 2
_NS = 16
_LANES = 16
_NW = _NC * _NS


def _sc_lookup_scale(weight):
    f_dim, c_dim = weight.shape
    rows_pw = f_dim // _NS
    per_row = c_dim // _LANES
    scale = _LR_SCALE * _WEIGHT_SCALE

    def body(w_hbm, out_hbm, v):
        wid = lax.axis_index("s")
        base = wid * rows_pw
        pltpu.sync_copy(w_hbm.at[pl.ds(base, rows_pw), :], v)
        for r in range(rows_pw):
            for k in range(per_row):
                sl = pl.ds(k * _LANES, _LANES)
                v[r, sl] = v[r, sl] * scale
        pltpu.sync_copy(v, out_hbm.at[pl.ds(base, rows_pw), :])

    mesh = plsc.VectorSubcoreMesh(
        core_axis_name="c", subcore_axis_name="s", num_cores=1
    )
    fn = functools.partial(
        pl.kernel,
        mesh=mesh,
        out_type=jax.ShapeDtypeStruct((f_dim, c_dim), jnp.float32),
        scratch_types=[pltpu.VMEM((rows_pw, c_dim), jnp.float32)],
    )(body)
    return fn(weight)


_NGRP = 8


def _tc_expand(tbl, batch, f_dim, c_dim, t_dim):
    rows = c_dim * f_dim
    cpg = c_dim // _NGRP

    def body(t_ref, out_ref, scratch_ref, sem):
        t = t_ref[...]
        copies = []
        for g in range(_NGRP):
            for c in range(g * cpg, (g + 1) * cpg):
                col = lax.slice(t, (0, c), (f_dim, c + 1))
                scratch_ref[pl.ds(c * f_dim, f_dim), :] = lax.broadcast_in_dim(
                    col, (f_dim, t_dim), (0, 1)
                )
            sl = pl.ds(g * cpg * f_dim, cpg * f_dim)
            for b in range(batch):
                cp = pltpu.make_async_copy(
                    scratch_ref.at[sl, :], out_ref.at[b, sl, :], sem
                )
                cp.start()
                copies.append(cp)
        for cp in copies:
            cp.wait()

    return pl.pallas_call(
        body,
        in_specs=[pl.BlockSpec(memory_space=pltpu.VMEM)],
        out_specs=pl.BlockSpec(memory_space=pl.ANY),
        out_shape=jax.ShapeDtypeStruct((batch, rows, t_dim), jnp.float32),
        scratch_shapes=[
            pltpu.VMEM((rows, t_dim), jnp.float32),
            pltpu.SemaphoreType.DMA,
        ],
        compiler_params=pltpu.CompilerParams(
            vmem_limit_bytes=100 * 1024 * 1024,
        ),
    )(tbl)


def kernel(input_features, weight):
    batch, c_dim, f_dim, t_dim = input_features.shape
    tbl = _sc_lookup_scale(weight)
    out3 = _tc_expand(tbl, batch, f_dim, c_dim, t_dim)
    return out3.reshape(batch, c_dim, f_dim, t_dim)


# speedup vs baseline: 1.0217x; 1.0091x over previous
import functools

import jax
import jax.numpy as jnp
from jax import lax
from jax.experimental import pallas as pl
from jax.experimental.pallas import tpu as pltpu
from jax.experimental.pallas import tpu_sc as plsc

_LR_SCALE = 10.0
_WEIGHT_SCALE = 0.2

_NS = 16
_LANES = 16


def _sc_lookup_scale(weight):
    f_dim, c_dim = weight.shape
    rows_pw = f_dim // _NS
    per_row = c_dim // _LANES
    scale = _LR_SCALE * _WEIGHT_SCALE

    def body(w_hbm, out_hbm, v):
        wid = lax.axis_index("s")
        base = wid * rows_pw
        pltpu.sync_copy(w_hbm.at[pl.ds(base, rows_pw), :], v)
        for r in range(rows_pw):
            for k in range(per_row):
                sl = pl.ds(k * _LANES, _LANES)
                v[r, sl] = v[r, sl] * scale
        pltpu.sync_copy(v, out_hbm.at[pl.ds(base, rows_pw), :])

    mesh = plsc.VectorSubcoreMesh(
        core_axis_name="c", subcore_axis_name="s", num_cores=1
    )
    fn = functools.partial(
        pl.kernel,
        mesh=mesh,
        out_type=jax.ShapeDtypeStruct((f_dim, c_dim), jnp.float32),
        scratch_types=[pltpu.VMEM((rows_pw, c_dim), jnp.float32)],
    )(body)
    return fn(weight)


_NGRP = 8


def _tc_expand(tbl, batch, f_dim, c_dim, t_dim):
    rows = c_dim * f_dim
    cpg = c_dim // _NGRP

    def body(t_ref, out_ref, scratch_ref, sem):
        t = t_ref[...]
        copies = []
        for g in range(_NGRP):
            for c in range(g * cpg, (g + 1) * cpg):
                col = lax.slice(t, (0, c), (f_dim, c + 1))
                scratch_ref[pl.ds(c * f_dim, f_dim), :] = lax.broadcast_in_dim(
                    col, (f_dim, t_dim), (0, 1)
                )
            sl = pl.ds(g * cpg * f_dim, cpg * f_dim)
            for b in range(batch):
                cp = pltpu.make_async_copy(
                    scratch_ref.at[sl, :], out_ref.at[b, sl, :], sem
                )
                cp.start()
                copies.append(cp)
        for cp in copies:
            cp.wait()

    return pl.pallas_call(
        body,
        in_specs=[pl.BlockSpec(memory_space=pltpu.VMEM)],
        out_specs=pl.BlockSpec(memory_space=pl.ANY),
        out_shape=jax.ShapeDtypeStruct((batch, rows, t_dim), jnp.float32),
        scratch_shapes=[
            pltpu.VMEM((rows, t_dim), jnp.float32),
            pltpu.SemaphoreType.DMA,
        ],
        compiler_params=pltpu.CompilerParams(
            vmem_limit_bytes=100 * 1024 * 1024,
        ),
    )(tbl)


def kernel(input_features, weight):
    batch, c_dim, f_dim, t_dim = input_features.shape
    tbl = _sc_lookup_scale(weight)
    out3 = _tc_expand(tbl, batch, f_dim, c_dim, t_dim)
    return out3.reshape(batch, c_dim, f_dim, t_dim)
